# baseline (speedup 1.0000x reference)
import functools

import jax
import jax.numpy as jnp
from jax import lax
from jax.experimental import pallas as pl
from jax.experimental.pallas import tpu as pltpu
from jax.experimental.pallas import tpu_sc as plsc

VOCAB = 8192
D = 8192
BATCH = 4096

NC = 2
NS = 16
NW = NC * NS
B_PER_W = BATCH // NW
GRP = 16
N_GRP = B_PER_W // GRP

_mesh = plsc.VectorSubcoreMesh(core_axis_name="c", subcore_axis_name="s")


@functools.partial(
    pl.kernel,
    mesh=_mesh,
    out_type=jax.ShapeDtypeStruct((BATCH, D), jnp.float32),
    scratch_types=[
        pltpu.VMEM((B_PER_W,), jnp.int32),
        pltpu.SemaphoreType.DMA,
    ],
)
def _gather_rows(x_hbm, table_hbm, out_hbm, idx_v, sem):
    wid = lax.axis_index("s") * NC + lax.axis_index("c")
    row0 = wid * B_PER_W

    pltpu.sync_copy(x_hbm.at[pl.ds(row0, B_PER_W)], idx_v)

    def fire(g):
        vec = idx_v[pl.ds(g * GRP, GRP)]
        for j in range(GRP):
            s = vec[j]
            pltpu.make_async_copy(
                table_hbm.at[pl.ds(s, 1)],
                out_hbm.at[pl.ds(row0 + g * GRP + j, 1)],
                sem,
            ).start()

    def drain(g):
        pltpu.make_async_copy(
            table_hbm.at[pl.ds(0, GRP)],
            out_hbm.at[pl.ds(row0, GRP)],
            sem,
        ).wait()

    fire(0)

    def body(g, carry):
        fire(g)
        drain(g - 1)
        return carry

    lax.fori_loop(1, N_GRP, body, 0)
    drain(N_GRP - 1)


def kernel(x, probs):
    return _gather_rows(x.astype(jnp.int32), probs)


# speedup vs baseline: 36.2770x; 36.2770x over previous
import functools

import jax
import jax.numpy as jnp
from jax import lax
from jax.experimental import pallas as pl
from jax.experimental.pallas import tpu as pltpu
from jax.experimental.pallas import tpu_sc as plsc

VOCAB = 8192
D = 8192
BATCH = 4096

NC = 2
NS = 16
NW = NC * NS
B_PER_W = BATCH // NW
GRP = 16
N_GRP = B_PER_W // GRP
QD = 2048
NQ = D // QD
NBUF = 3

_mesh = plsc.VectorSubcoreMesh(core_axis_name="c", subcore_axis_name="s")


@functools.partial(
    pl.kernel,
    mesh=_mesh,
    out_type=jax.ShapeDtypeStruct((BATCH, D), jnp.float32),
    scratch_types=[
        pltpu.VMEM((B_PER_W,), jnp.int32),
        pltpu.VMEM((GRP, QD), jnp.float32),
        pltpu.VMEM((GRP, QD), jnp.float32),
        pltpu.VMEM((GRP, QD), jnp.float32),
        pltpu.SemaphoreType.DMA,
        pltpu.SemaphoreType.DMA,
        pltpu.SemaphoreType.DMA,
        pltpu.SemaphoreType.DMA,
        pltpu.SemaphoreType.DMA,
        pltpu.SemaphoreType.DMA,
    ],
)
def _gather_rows(x_hbm, table_hbm, out_hbm, idx_v, buf0, buf1, buf2,
                 g0, g1, g2, w0, w1, w2):
    wid = lax.axis_index("s") * NC + lax.axis_index("c")
    row0 = wid * B_PER_W
    bufs = (buf0, buf1, buf2)
    gsems = (g0, g1, g2)
    wsems = (w0, w1, w2)

    pltpu.sync_copy(x_hbm.at[pl.ds(row0, B_PER_W)], idx_v)

    N_STEP = N_GRP * NQ

    def start_gather(t, b):
        g = t // NQ
        q = t % NQ
        vec = idx_v[pl.ds(g * GRP, GRP)]
        pltpu.make_async_copy(
            table_hbm.at[vec, pl.ds(q * QD, QD)], bufs[b], gsems[b]).start()

    def wait_gather(b):
        vec0 = idx_v[pl.ds(0, GRP)]
        pltpu.make_async_copy(
            table_hbm.at[vec0, pl.ds(0, QD)], bufs[b], gsems[b]).wait()

    def start_write(t, b):
        g = t // NQ
        q = t % NQ
        pltpu.make_async_copy(
            bufs[b],
            out_hbm.at[pl.ds(row0 + g * GRP, GRP), pl.ds(q * QD, QD)],
            wsems[b]).start()

    def wait_write(b):
        pltpu.make_async_copy(
            bufs[b], out_hbm.at[pl.ds(row0, GRP), pl.ds(0, QD)], wsems[b]).wait()

    def emit(t, b):
        bp = (b + NBUF - 1) % NBUF
        wait_gather(b)
        start_write(t, b)
        if isinstance(t, int):
            if t > 0:
                wait_write(bp)
            if t + NBUF - 1 < N_STEP:
                start_gather(t + NBUF - 1, bp)
        else:
            @pl.when(t > 0)
            def _retire():
                wait_write(bp)

            @pl.when(t + NBUF - 1 < N_STEP)
            def _prefetch():
                start_gather(t + NBUF - 1, bp)

    for b in range(NBUF - 1):
        start_gather(b, b)

    N_MAIN = (N_STEP // NBUF) * NBUF

    def outer(gg, carry):
        for b in range(NBUF):
            emit(gg * NBUF + b, b)
        return carry

    lax.fori_loop(0, N_MAIN // NBUF, outer, 0)
    for t in range(N_MAIN, N_STEP):
        emit(t, t % NBUF)
    wait_write((N_STEP - 1) % NBUF)


def kernel(x, probs):
    return _gather_rows(x.astype(jnp.int32), probs)


# speedup vs baseline: 36.4065x; 1.0036x over previous
import functools

import jax
import jax.numpy as jnp
from jax import lax
from jax.experimental import pallas as pl
from jax.experimental.pallas import tpu as pltpu
from jax.experimental.pallas import tpu_sc as plsc

VOCAB = 8192
D = 8192
BATCH = 4096

NC = 2
NS = 16
NW = NC * NS
B_PER_W = BATCH // NW
CHUNK = 2
NBUF = 3
N_CHUNKS = B_PER_W // CHUNK
N_MAIN = (N_CHUNKS // NBUF) * NBUF

_mesh = plsc.VectorSubcoreMesh(core_axis_name="c", subcore_axis_name="s")


@functools.partial(
    pl.kernel,
    mesh=_mesh,
    out_type=jax.ShapeDtypeStruct((BATCH, D), jnp.float32),
    scratch_types=[
        pltpu.VMEM((N_CHUNKS, CHUNK), jnp.int32),
        pltpu.VMEM((CHUNK, D), jnp.float32),
        pltpu.VMEM((CHUNK, D), jnp.float32),
        pltpu.VMEM((CHUNK, D), jnp.float32),
        pltpu.SemaphoreType.DMA,
        pltpu.SemaphoreType.DMA,
        pltpu.SemaphoreType.DMA,
        pltpu.SemaphoreType.DMA,
        pltpu.SemaphoreType.DMA,
        pltpu.SemaphoreType.DMA,
    ],
)
def _gather_rows(x_hbm, table_hbm, out_hbm, idx_v, buf0, buf1, buf2,
                 g0, g1, g2, w0, w1, w2):
    wid = lax.axis_index("s") * NC + lax.axis_index("c")
    row0 = wid * B_PER_W
    bufs = (buf0, buf1, buf2)
    gsems = (g0, g1, g2)
    wsems = (w0, w1, w2)

    pltpu.sync_copy(x_hbm.at[wid], idx_v)

    def start_gather(c, b):
        pltpu.make_async_copy(table_hbm.at[idx_v.at[c]], bufs[b], gsems[b]).start()

    def wait_gather(b):
        pltpu.make_async_copy(table_hbm.at[idx_v.at[0]], bufs[b], gsems[b]).wait()

    def start_write(c, b):
        pltpu.make_async_copy(
            bufs[b], out_hbm.at[pl.ds(row0 + c * CHUNK, CHUNK)], wsems[b]).start()

    def wait_write(b):
        pltpu.make_async_copy(
            bufs[b], out_hbm.at[pl.ds(row0, CHUNK)], wsems[b]).wait()

    def emit(c, b):
        bp = (b + NBUF - 1) % NBUF
        wait_gather(b)
        start_write(c, b)
        if isinstance(c, int):
            if c > 0:
                wait_write(bp)
            if c + NBUF - 1 < N_CHUNKS:
                start_gather(c + NBUF - 1, bp)
        else:
            @pl.when(c > 0)
            def _retire():
                wait_write(bp)

            @pl.when(c + NBUF - 1 < N_CHUNKS)
            def _prefetch():
                start_gather(c + NBUF - 1, bp)

    for b in range(NBUF - 1):
        start_gather(b, b)

    def outer(g, carry):
        for b in range(NBUF):
            emit(g * NBUF + b, b)
        return carry

    lax.fori_loop(0, N_MAIN // NBUF, outer, 0)
    for c in range(N_MAIN, N_CHUNKS):
        emit(c, c % NBUF)
    wait_write((N_CHUNKS - 1) % NBUF)


def kernel(x, probs):
    x_chunked = x.astype(jnp.int32).reshape(NW, N_CHUNKS, CHUNK)
    return _gather_rows(x_chunked, probs)
